# baseline (speedup 1.0000x reference)
import jax
import jax.numpy as jnp
from jax.experimental import pallas as pl
from jax.experimental.pallas import tpu as pltpu


def _add_kernel(x_ref, pos_ref, out_ref):
    out_ref[...] = x_ref[...] + pos_ref[...]


def kernel(x, pos_emb_weight):
    Bx, Tx, Dx = x.shape
    RB = 1024
    DB = 1024
    n_chunks = Tx // RB
    n_d = Dx // DB
    xf = x.reshape(Bx * Tx, Dx)
    out = pl.pallas_call(
        _add_kernel,
        grid=(n_chunks, n_d, Bx),
        in_specs=[
            pl.BlockSpec((RB, DB), lambda p, d, b: (b * n_chunks + p, d)),
            pl.BlockSpec((RB, DB), lambda p, d, b: (p, d)),
        ],
        out_specs=pl.BlockSpec((RB, DB), lambda p, d, b: (b * n_chunks + p, d)),
        out_shape=jax.ShapeDtypeStruct((Bx * Tx, Dx), x.dtype),
        compiler_params=pltpu.CompilerParams(
            dimension_semantics=("arbitrary", "arbitrary", "arbitrary"),
        ),
    )(xf, pos_emb_weight[:Tx])
    return out.reshape(Bx, Tx, Dx)


# speedup vs baseline: 1.0453x; 1.0453x over previous
import jax
import jax.numpy as jnp
from jax.experimental import pallas as pl
from jax.experimental.pallas import tpu as pltpu


def _add_kernel(x_ref, pos_ref, out_ref):
    out_ref[...] = x_ref[...] + pos_ref[...]


def kernel(x, pos_emb_weight):
    Bx, Tx, Dx = x.shape
    RB = 2048
    n_chunks = Tx // RB
    xf = x.reshape(Bx * Tx, Dx)
    out = pl.pallas_call(
        _add_kernel,
        grid=(n_chunks, Bx),
        in_specs=[
            pl.BlockSpec((RB, Dx), lambda p, b: (b * n_chunks + p, 0)),
            pl.BlockSpec((RB, Dx), lambda p, b: (p, 0)),
        ],
        out_specs=pl.BlockSpec((RB, Dx), lambda p, b: (b * n_chunks + p, 0)),
        out_shape=jax.ShapeDtypeStruct((Bx * Tx, Dx), x.dtype),
        compiler_params=pltpu.CompilerParams(
            dimension_semantics=("arbitrary", "arbitrary"),
        ),
    )(xf, pos_emb_weight[:Tx])
    return out.reshape(Bx, Tx, Dx)
